# Initial kernel scaffold; baseline (speedup 1.0000x reference)
import jax
import jax.numpy as jnp
from jax.experimental import pallas as pl


def kernel(x, edge_index, batch, W_in, b_in, conv_W, conv_b, ln_g, ln_b, W1, b1, W2, b2):
    raise NotImplementedError("write your pallas kernel here")


# speedup vs baseline: 12.7745x; 12.7745x over previous
import functools

import jax
import jax.numpy as jnp
from jax import lax
from jax.experimental import pallas as pl
from jax.experimental.pallas import tpu as pltpu
from jax.experimental.pallas import tpu_sc as plsc

N = 10000
E = 320000
H = 128
G = 128
L = 3

NC = 2
NS = 16
NW = NC * NS
LANES = 16

CH = 128
NCHUNKS = E // CH

CHZ = 200
NZCHUNKS = N // CHZ

CHP = 200
NPCHUNKS = N // CHP
G_PER_SUB = G // NS

def _fill_rows(buf, nrows, ngroups, val):
    v = jnp.full((LANES,), val, dtype=buf.dtype)

    @pl.loop(0, nrows)
    def _(r):
        @pl.loop(0, ngroups)
        def _(c):
            buf[r, pl.ds(c * LANES, LANES)] = v


@functools.lru_cache(maxsize=None)
def _sc_kernels():
    mesh = plsc.VectorSubcoreMesh(core_axis_name="c", subcore_axis_name="s")

    @functools.partial(
        pl.kernel,
        out_type=jax.ShapeDtypeStruct((NC, N, H), jnp.float32),
        mesh=mesh,
        scratch_types=[
            pltpu.VMEM((CH,), jnp.int32),
            pltpu.VMEM((CHZ, H), jnp.float32),
            pltpu.VMEM_SHARED((N, H), jnp.float32),
        ],
    )
    def sc_degree(dst_hbm, out_hbm, dst_v, ones_v, deg_sh):
        _sc_degree_body(dst_hbm, out_hbm, dst_v, ones_v, deg_sh)

    @functools.partial(
        pl.kernel,
        out_type=jax.ShapeDtypeStruct((NC, N, H), jnp.float32),
        mesh=mesh,
        scratch_types=[
            pltpu.VMEM((CH,), jnp.int32),
            pltpu.VMEM((CH,), jnp.int32),
            pltpu.VMEM((CH, H), jnp.float32),
            pltpu.VMEM((CHZ, H), jnp.float32),
            pltpu.VMEM_SHARED((N, H), jnp.float32),
        ],
    )
    def sc_edge_agg(m2_hbm, src_hbm, dst_hbm, out_hbm, src_v, dst_v, rows_v,
                    zero_v, agg_sh):
        _sc_edge_agg_body(m2_hbm, src_hbm, dst_hbm, out_hbm, src_v, dst_v,
                          rows_v, zero_v, agg_sh)

    @functools.partial(
        pl.kernel,
        out_type=[
            jax.ShapeDtypeStruct((NC, G, H), jnp.float32),
            jax.ShapeDtypeStruct((NC, G, H), jnp.float32),
        ],
        mesh=mesh,
        scratch_types=[
            pltpu.VMEM((CHP,), jnp.int32),
            pltpu.VMEM((CHP, H), jnp.float32),
            pltpu.VMEM((CHP, H), jnp.float32),
            pltpu.VMEM_SHARED((G, H), jnp.float32),
            pltpu.VMEM_SHARED((G, H), jnp.float32),
        ],
    )
    def sc_pool(h_hbm, batch_hbm, sum_hbm, cnt_hbm, batch_v, rows_v, ones_v,
                sum_sh, cnt_sh):
        _sc_pool_body(h_hbm, batch_hbm, sum_hbm, cnt_hbm, batch_v, rows_v,
                      ones_v, sum_sh, cnt_sh)

    return sc_degree, sc_edge_agg, sc_pool


def _sc_degree_body(dst_hbm, out_hbm, dst_v, ones_v, deg_sh):
    cid = lax.axis_index("c")
    sid = lax.axis_index("s")
    wid = sid * NC + cid

    _fill_rows(ones_v, CHZ, H // LANES, 0.0)

    @pl.loop(0, NZCHUNKS // NS + 1)
    def _(t):
        zc = sid + NS * t

        @pl.when(zc < NZCHUNKS)
        def _():
            pltpu.sync_copy(ones_v, deg_sh.at[pl.ds(zc * CHZ, CHZ)])

    plsc.subcore_barrier()
    _fill_rows(ones_v, CH, H // LANES, 1.0)

    nper = NCHUNKS // NW

    @pl.loop(0, nper + 1)
    def _(j):
        chunk = wid + NW * j

        @pl.when(chunk < NCHUNKS)
        def _():
            pltpu.sync_copy(dst_hbm.at[pl.ds(chunk * CH, CH)], dst_v)
            pltpu.sync_copy(ones_v.at[pl.ds(0, CH)], deg_sh.at[dst_v],
                            add=True)

    plsc.subcore_barrier()

    @pl.loop(0, NZCHUNKS // NS + 1)
    def _(t):
        zc = sid + NS * t

        @pl.when(zc < NZCHUNKS)
        def _():
            pltpu.sync_copy(deg_sh.at[pl.ds(zc * CHZ, CHZ)],
                            out_hbm.at[cid, pl.ds(zc * CHZ, CHZ)])


def _sc_edge_agg_body(m2_hbm, src_hbm, dst_hbm, out_hbm, src_v, dst_v, rows_v,
                      zero_v, agg_sh):
    cid = lax.axis_index("c")
    sid = lax.axis_index("s")
    wid = sid * NC + cid

    _fill_rows(zero_v, CHZ, H // LANES, 0.0)

    @pl.loop(0, NZCHUNKS // NS + 1)
    def _(t):
        zc = sid + NS * t

        @pl.when(zc < NZCHUNKS)
        def _():
            pltpu.sync_copy(zero_v, agg_sh.at[pl.ds(zc * CHZ, CHZ)])

    plsc.subcore_barrier()

    nper = NCHUNKS // NW

    @pl.loop(0, nper + 1)
    def _(j):
        chunk = wid + NW * j

        @pl.when(chunk < NCHUNKS)
        def _():
            base = chunk * CH
            pltpu.sync_copy(src_hbm.at[pl.ds(base, CH)], src_v)
            pltpu.sync_copy(dst_hbm.at[pl.ds(base, CH)], dst_v)
            pltpu.sync_copy(m2_hbm.at[src_v], rows_v)
            pltpu.sync_copy(rows_v, agg_sh.at[dst_v], add=True)

    plsc.subcore_barrier()

    @pl.loop(0, NZCHUNKS // NS + 1)
    def _(t):
        zc = sid + NS * t

        @pl.when(zc < NZCHUNKS)
        def _():
            pltpu.sync_copy(agg_sh.at[pl.ds(zc * CHZ, CHZ)],
                            out_hbm.at[cid, pl.ds(zc * CHZ, CHZ)])


def _sc_pool_body(h_hbm, batch_hbm, sum_hbm, cnt_hbm, batch_v, rows_v, ones_v,
                  sum_sh, cnt_sh):
    cid = lax.axis_index("c")
    sid = lax.axis_index("s")
    wid = sid * NC + cid

    _fill_rows(rows_v, G_PER_SUB, H // LANES, 0.0)
    _fill_rows(ones_v, G_PER_SUB, H // LANES, 0.0)
    pltpu.sync_copy(rows_v.at[pl.ds(0, G_PER_SUB)],
                    sum_sh.at[pl.ds(sid * G_PER_SUB, G_PER_SUB)])
    pltpu.sync_copy(ones_v.at[pl.ds(0, G_PER_SUB)],
                    cnt_sh.at[pl.ds(sid * G_PER_SUB, G_PER_SUB)])
    plsc.subcore_barrier()
    _fill_rows(ones_v, CHP, H // LANES, 1.0)

    nper = NPCHUNKS // NW

    @pl.loop(0, nper + 1)
    def _(j):
        chunk = wid + NW * j

        @pl.when(chunk < NPCHUNKS)
        def _():
            base = chunk * CHP
            pltpu.sync_copy(h_hbm.at[pl.ds(base, CHP)], rows_v)
            pltpu.sync_copy(batch_hbm.at[pl.ds(base, CHP)], batch_v)
            pltpu.sync_copy(rows_v, sum_sh.at[batch_v], add=True)
            pltpu.sync_copy(ones_v, cnt_sh.at[batch_v], add=True)

    plsc.subcore_barrier()
    pltpu.sync_copy(sum_sh.at[pl.ds(sid * G_PER_SUB, G_PER_SUB)],
                    sum_hbm.at[cid, pl.ds(sid * G_PER_SUB, G_PER_SUB)])
    pltpu.sync_copy(cnt_sh.at[pl.ds(sid * G_PER_SUB, G_PER_SUB)],
                    cnt_hbm.at[cid, pl.ds(sid * G_PER_SUB, G_PER_SUB)])


_BN = 1000


def _dinv_body(d_ref, o_ref):
    deg = d_ref[0, :, 0:1] + d_ref[1, :, 0:1] + 1.0
    o_ref[...] = lax.rsqrt(deg)


def _tc_dinv(deg_parts):
    return pl.pallas_call(
        _dinv_body,
        grid=(N // _BN,),
        in_specs=[pl.BlockSpec((NC, _BN, H), lambda i: (0, i, 0))],
        out_specs=pl.BlockSpec((_BN, 1), lambda i: (i, 0)),
        out_shape=jax.ShapeDtypeStruct((N, 1), jnp.float32),
    )(deg_parts)


def _mm_body(h_ref, w_ref, b_ref, s_ref, o_ref):
    acc = jnp.dot(h_ref[...], w_ref[...], preferred_element_type=jnp.float32)
    o_ref[...] = (acc + b_ref[...]) * s_ref[...]


def _tc_mm(h, w, bias, scale):
    return pl.pallas_call(
        _mm_body,
        grid=(N // _BN,),
        in_specs=[
            pl.BlockSpec((_BN, H), lambda i: (i, 0)),
            pl.BlockSpec((H, H), lambda i: (0, 0)),
            pl.BlockSpec((1, H), lambda i: (0, 0)),
            pl.BlockSpec((_BN, 1), lambda i: (i, 0)),
        ],
        out_specs=pl.BlockSpec((_BN, H), lambda i: (i, 0)),
        out_shape=jax.ShapeDtypeStruct((N, H), jnp.float32),
    )(h, w, bias, scale)


def _post_body(a_ref, m_ref, s_ref, cb_ref, g_ref, b_ref, hp_ref, o_ref):
    t = (a_ref[0] + a_ref[1] + m_ref[...]) * s_ref[...] + cb_ref[...]
    mu = jnp.mean(t, axis=-1, keepdims=True)
    var = jnp.mean((t - mu) ** 2, axis=-1, keepdims=True)
    hn = (t - mu) * lax.rsqrt(var + 1e-5) * g_ref[...] + b_ref[...]
    o_ref[...] = jnp.maximum(hn, 0.0) + hp_ref[...]


def _tc_post(agg_parts, m2, dinv, conv_b_i, ln_g_i, ln_b_i, h_prev):
    return pl.pallas_call(
        _post_body,
        grid=(N // _BN,),
        in_specs=[
            pl.BlockSpec((NC, _BN, H), lambda i: (0, i, 0)),
            pl.BlockSpec((_BN, H), lambda i: (i, 0)),
            pl.BlockSpec((_BN, 1), lambda i: (i, 0)),
            pl.BlockSpec((1, H), lambda i: (0, 0)),
            pl.BlockSpec((1, H), lambda i: (0, 0)),
            pl.BlockSpec((1, H), lambda i: (0, 0)),
            pl.BlockSpec((_BN, H), lambda i: (i, 0)),
        ],
        out_specs=pl.BlockSpec((_BN, H), lambda i: (i, 0)),
        out_shape=jax.ShapeDtypeStruct((N, H), jnp.float32),
    )(agg_parts, m2, dinv, conv_b_i, ln_g_i, ln_b_i, h_prev)


def _mlp_body(pp_ref, cp_ref, w1_ref, b1_ref, w2_ref, b2_ref, o_ref):
    sums = pp_ref[0] + pp_ref[1]
    cnts = cp_ref[0, :, 0:1] + cp_ref[1, :, 0:1]
    pooled = sums / jnp.maximum(cnts, 1.0)
    z = jnp.dot(pooled, w1_ref[...], preferred_element_type=jnp.float32)
    z = jnp.maximum(z + b1_ref[...], 0.0)
    o_ref[...] = jnp.dot(z, w2_ref[...],
                         preferred_element_type=jnp.float32) + b2_ref[...]


def _tc_mlp(pool_parts, cnt_parts, w1, b1, w2, b2):
    h2 = w1.shape[1]
    return pl.pallas_call(
        _mlp_body,
        grid=(1,),
        in_specs=[
            pl.BlockSpec((NC, G, H), lambda i: (0, 0, 0)),
            pl.BlockSpec((NC, G, H), lambda i: (0, 0, 0)),
            pl.BlockSpec((H, h2), lambda i: (0, 0)),
            pl.BlockSpec((1, h2), lambda i: (0, 0)),
            pl.BlockSpec((h2, 1), lambda i: (0, 0)),
            pl.BlockSpec((1, 1), lambda i: (0, 0)),
        ],
        out_specs=pl.BlockSpec((G, 1), lambda i: (0, 0)),
        out_shape=jax.ShapeDtypeStruct((G, 1), jnp.float32),
    )(pool_parts, cnt_parts, w1, b1, w2, b2)


def kernel(x, edge_index, batch, W_in, b_in, conv_W, conv_b, ln_g, ln_b,
           W1, b1, W2, b2):
    src = edge_index[0]
    dst = edge_index[1]
    _sc_degree, _sc_edge_agg, _sc_pool = _sc_kernels()

    deg_parts = _sc_degree(dst)
    dinv = _tc_dinv(deg_parts)

    ones_scale = jnp.ones((N, 1), jnp.float32)
    h = _tc_mm(x, W_in, b_in.reshape(1, H), ones_scale)

    zero_bias = jnp.zeros((1, H), jnp.float32)
    for i in range(L):
        m2 = _tc_mm(h, conv_W[i], zero_bias, dinv)
        agg_parts = _sc_edge_agg(m2, src, dst)
        h = _tc_post(agg_parts, m2, dinv, conv_b[i].reshape(1, H),
                     ln_g[i].reshape(1, H), ln_b[i].reshape(1, H), h)

    pool_parts, cnt_parts = _sc_pool(h, batch)
    out = _tc_mlp(pool_parts, cnt_parts, W1, b1.reshape(1, -1), W2,
                  b2.reshape(1, 1))
    return out


# speedup vs baseline: 21.4553x; 1.6795x over previous
import functools

import jax
import jax.numpy as jnp
from jax import lax
from jax.experimental import pallas as pl
from jax.experimental.pallas import tpu as pltpu
from jax.experimental.pallas import tpu_sc as plsc

N = 10000
E = 320000
H = 128
G = 128
L = 3

NC = 2
NS = 16
NW = NC * NS
LANES = 16

CH = 128
NCHUNKS = E // CH
NBUF = 3

CHZ = 200
NZCHUNKS = N // CHZ

CHP = 200
NPCHUNKS = N // CHP
G_PER_SUB = G // NS

def _fill_rows(buf, nrows, ngroups, val):
    v = jnp.full((LANES,), val, dtype=buf.dtype)

    @pl.loop(0, nrows)
    def _(r):
        @pl.loop(0, ngroups)
        def _(c):
            buf[r, pl.ds(c * LANES, LANES)] = v


@functools.lru_cache(maxsize=None)
def _sc_kernels():
    mesh = plsc.VectorSubcoreMesh(core_axis_name="c", subcore_axis_name="s")

    @functools.partial(
        pl.kernel,
        out_type=jax.ShapeDtypeStruct((NC, N, H), jnp.float32),
        mesh=mesh,
        scratch_types=[
            pltpu.VMEM((CH,), jnp.int32),
            pltpu.VMEM((CHZ, H), jnp.float32),
            pltpu.VMEM_SHARED((N, H), jnp.float32),
        ],
    )
    def sc_degree(dst_hbm, out_hbm, dst_v, ones_v, deg_sh):
        _sc_degree_body(dst_hbm, out_hbm, dst_v, ones_v, deg_sh)

    @functools.partial(
        pl.kernel,
        out_type=jax.ShapeDtypeStruct((NC, N, H), jnp.float32),
        mesh=mesh,
        scratch_types=(
            [pltpu.VMEM((2, CH), jnp.int32) for _ in range(NBUF)]
            + [pltpu.VMEM((CH, H), jnp.float32) for _ in range(NBUF)]
            + [pltpu.SemaphoreType.DMA for _ in range(NBUF)]
            + [pltpu.VMEM_SHARED((N, H), jnp.float32)]
        ),
    )
    def sc_edge_agg(m2_hbm, edge_hbm, out_hbm, *refs):
        idx_bufs = refs[0:NBUF]
        row_bufs = refs[NBUF:2 * NBUF]
        sems = refs[2 * NBUF:3 * NBUF]
        agg_sh = refs[3 * NBUF]
        _sc_edge_agg_body(m2_hbm, edge_hbm, out_hbm, idx_bufs, row_bufs,
                          sems, agg_sh)

    @functools.partial(
        pl.kernel,
        out_type=[
            jax.ShapeDtypeStruct((NC, G, H), jnp.float32),
            jax.ShapeDtypeStruct((NC, G, H), jnp.float32),
        ],
        mesh=mesh,
        scratch_types=[
            pltpu.VMEM((CHP,), jnp.int32),
            pltpu.VMEM((CHP, H), jnp.float32),
            pltpu.VMEM((CHP, H), jnp.float32),
            pltpu.VMEM_SHARED((G, H), jnp.float32),
            pltpu.VMEM_SHARED((G, H), jnp.float32),
        ],
    )
    def sc_pool(h_hbm, batch_hbm, sum_hbm, cnt_hbm, batch_v, rows_v, ones_v,
                sum_sh, cnt_sh):
        _sc_pool_body(h_hbm, batch_hbm, sum_hbm, cnt_hbm, batch_v, rows_v,
                      ones_v, sum_sh, cnt_sh)

    return sc_degree, sc_edge_agg, sc_pool


def _sc_degree_body(dst_hbm, out_hbm, dst_v, ones_v, deg_sh):
    cid = lax.axis_index("c")
    sid = lax.axis_index("s")
    wid = sid * NC + cid

    _fill_rows(ones_v, CHZ, H // LANES, 0.0)

    @pl.loop(0, NZCHUNKS // NS + 1)
    def _(t):
        zc = sid + NS * t

        @pl.when(zc < NZCHUNKS)
        def _():
            pltpu.sync_copy(ones_v, deg_sh.at[pl.ds(zc * CHZ, CHZ)])

    plsc.subcore_barrier()
    _fill_rows(ones_v, CH, H // LANES, 1.0)

    nper = NCHUNKS // NW

    @pl.loop(0, nper + 1)
    def _(j):
        chunk = wid + NW * j

        @pl.when(chunk < NCHUNKS)
        def _():
            pltpu.sync_copy(dst_hbm.at[pl.ds(chunk * CH, CH)], dst_v)
            pltpu.sync_copy(ones_v.at[pl.ds(0, CH)], deg_sh.at[dst_v],
                            add=True)

    plsc.subcore_barrier()

    @pl.loop(0, NZCHUNKS // NS + 1)
    def _(t):
        zc = sid + NS * t

        @pl.when(zc < NZCHUNKS)
        def _():
            pltpu.sync_copy(deg_sh.at[pl.ds(zc * CHZ, CHZ)],
                            out_hbm.at[cid, pl.ds(zc * CHZ, CHZ)])


def _sc_edge_agg_body(m2_hbm, edge_hbm, out_hbm, idx_bufs, row_bufs, sems,
                      agg_sh):
    cid = lax.axis_index("c")
    sid = lax.axis_index("s")
    wid = sid * NC + cid

    _fill_rows(row_bufs[0], CH, H // LANES, 0.0)

    @pl.loop(0, 5)
    def _(t):
        zc = sid + NS * t

        @pl.when(zc < N // CH)
        def _():
            pltpu.sync_copy(row_bufs[0], agg_sh.at[pl.ds(zc * CH, CH)])

    @pl.when(sid == 0)
    def _():
        pltpu.sync_copy(row_bufs[0].at[pl.ds(0, N % CH)],
                        agg_sh.at[pl.ds(N - N % CH, N % CH)])

    plsc.subcore_barrier()

    def gather_dma(b):
        return pltpu.make_async_copy(
            m2_hbm.at[idx_bufs[b].at[0]], row_bufs[b], sems[b])

    def issue(b, chunk):
        pltpu.sync_copy(edge_hbm.at[:, pl.ds(chunk * CH, CH)], idx_bufs[b])
        gather_dma(b).start()

    def finish(b):
        gather_dma(b).wait()
        pltpu.sync_copy(row_bufs[b], agg_sh.at[idx_bufs[b].at[1]], add=True)

    for b in range(NBUF):
        issue(b, wid + NW * b)

    nper_up = -(-NCHUNKS // NW)
    ngroups = -(-nper_up // NBUF) * NBUF

    @pl.loop(0, ngroups, step=NBUF)
    def _(j):
        for b in range(NBUF):
            chunk = wid + NW * (j + b)
            nxt = wid + NW * (j + b + NBUF)

            @pl.when(chunk < NCHUNKS)
            def _(b=b):
                finish(b)

            @pl.when(nxt < NCHUNKS)
            def _(b=b, nxt=nxt):
                issue(b, nxt)

    plsc.subcore_barrier()

    @pl.loop(0, NZCHUNKS // NS + 1)
    def _(t):
        zc = sid + NS * t

        @pl.when(zc < NZCHUNKS)
        def _():
            pltpu.sync_copy(agg_sh.at[pl.ds(zc * CHZ, CHZ)],
                            out_hbm.at[cid, pl.ds(zc * CHZ, CHZ)])


def _sc_pool_body(h_hbm, batch_hbm, sum_hbm, cnt_hbm, batch_v, rows_v, ones_v,
                  sum_sh, cnt_sh):
    cid = lax.axis_index("c")
    sid = lax.axis_index("s")
    wid = sid * NC + cid

    _fill_rows(rows_v, G_PER_SUB, H // LANES, 0.0)
    _fill_rows(ones_v, G_PER_SUB, H // LANES, 0.0)
    pltpu.sync_copy(rows_v.at[pl.ds(0, G_PER_SUB)],
                    sum_sh.at[pl.ds(sid * G_PER_SUB, G_PER_SUB)])
    pltpu.sync_copy(ones_v.at[pl.ds(0, G_PER_SUB)],
                    cnt_sh.at[pl.ds(sid * G_PER_SUB, G_PER_SUB)])
    plsc.subcore_barrier()
    _fill_rows(ones_v, CHP, H // LANES, 1.0)

    nper = NPCHUNKS // NW

    @pl.loop(0, nper + 1)
    def _(j):
        chunk = wid + NW * j

        @pl.when(chunk < NPCHUNKS)
        def _():
            base = chunk * CHP
            pltpu.sync_copy(h_hbm.at[pl.ds(base, CHP)], rows_v)
            pltpu.sync_copy(batch_hbm.at[pl.ds(base, CHP)], batch_v)
            pltpu.sync_copy(rows_v, sum_sh.at[batch_v], add=True)
            pltpu.sync_copy(ones_v, cnt_sh.at[batch_v], add=True)

    plsc.subcore_barrier()
    pltpu.sync_copy(sum_sh.at[pl.ds(sid * G_PER_SUB, G_PER_SUB)],
                    sum_hbm.at[cid, pl.ds(sid * G_PER_SUB, G_PER_SUB)])
    pltpu.sync_copy(cnt_sh.at[pl.ds(sid * G_PER_SUB, G_PER_SUB)],
                    cnt_hbm.at[cid, pl.ds(sid * G_PER_SUB, G_PER_SUB)])


_BN = 1000


def _dinv_body(d_ref, o_ref):
    deg = d_ref[0, :, 0:1] + d_ref[1, :, 0:1] + 1.0
    o_ref[...] = lax.rsqrt(deg)


def _tc_dinv(deg_parts):
    return pl.pallas_call(
        _dinv_body,
        grid=(N // _BN,),
        in_specs=[pl.BlockSpec((NC, _BN, H), lambda i: (0, i, 0))],
        out_specs=pl.BlockSpec((_BN, 1), lambda i: (i, 0)),
        out_shape=jax.ShapeDtypeStruct((N, 1), jnp.float32),
    )(deg_parts)


def _mm_body(h_ref, w_ref, b_ref, s_ref, o_ref):
    acc = jnp.dot(h_ref[...], w_ref[...], preferred_element_type=jnp.float32)
    o_ref[...] = (acc + b_ref[...]) * s_ref[...]


def _tc_mm(h, w, bias, scale):
    return pl.pallas_call(
        _mm_body,
        grid=(N // _BN,),
        in_specs=[
            pl.BlockSpec((_BN, H), lambda i: (i, 0)),
            pl.BlockSpec((H, H), lambda i: (0, 0)),
            pl.BlockSpec((1, H), lambda i: (0, 0)),
            pl.BlockSpec((_BN, 1), lambda i: (i, 0)),
        ],
        out_specs=pl.BlockSpec((_BN, H), lambda i: (i, 0)),
        out_shape=jax.ShapeDtypeStruct((N, H), jnp.float32),
    )(h, w, bias, scale)


def _post_body(a_ref, m_ref, s_ref, cb_ref, g_ref, b_ref, hp_ref, o_ref):
    t = (a_ref[0] + a_ref[1] + m_ref[...]) * s_ref[...] + cb_ref[...]
    mu = jnp.mean(t, axis=-1, keepdims=True)
    var = jnp.mean((t - mu) ** 2, axis=-1, keepdims=True)
    hn = (t - mu) * lax.rsqrt(var + 1e-5) * g_ref[...] + b_ref[...]
    o_ref[...] = jnp.maximum(hn, 0.0) + hp_ref[...]


def _tc_post(agg_parts, m2, dinv, conv_b_i, ln_g_i, ln_b_i, h_prev):
    return pl.pallas_call(
        _post_body,
        grid=(N // _BN,),
        in_specs=[
            pl.BlockSpec((NC, _BN, H), lambda i: (0, i, 0)),
            pl.BlockSpec((_BN, H), lambda i: (i, 0)),
            pl.BlockSpec((_BN, 1), lambda i: (i, 0)),
            pl.BlockSpec((1, H), lambda i: (0, 0)),
            pl.BlockSpec((1, H), lambda i: (0, 0)),
            pl.BlockSpec((1, H), lambda i: (0, 0)),
            pl.BlockSpec((_BN, H), lambda i: (i, 0)),
        ],
        out_specs=pl.BlockSpec((_BN, H), lambda i: (i, 0)),
        out_shape=jax.ShapeDtypeStruct((N, H), jnp.float32),
    )(agg_parts, m2, dinv, conv_b_i, ln_g_i, ln_b_i, h_prev)


def _mlp_body(pp_ref, cp_ref, w1_ref, b1_ref, w2_ref, b2_ref, o_ref):
    sums = pp_ref[0] + pp_ref[1]
    cnts = cp_ref[0, :, 0:1] + cp_ref[1, :, 0:1]
    pooled = sums / jnp.maximum(cnts, 1.0)
    z = jnp.dot(pooled, w1_ref[...], preferred_element_type=jnp.float32)
    z = jnp.maximum(z + b1_ref[...], 0.0)
    o_ref[...] = jnp.dot(z, w2_ref[...],
                         preferred_element_type=jnp.float32) + b2_ref[...]


def _tc_mlp(pool_parts, cnt_parts, w1, b1, w2, b2):
    h2 = w1.shape[1]
    return pl.pallas_call(
        _mlp_body,
        grid=(1,),
        in_specs=[
            pl.BlockSpec((NC, G, H), lambda i: (0, 0, 0)),
            pl.BlockSpec((NC, G, H), lambda i: (0, 0, 0)),
            pl.BlockSpec((H, h2), lambda i: (0, 0)),
            pl.BlockSpec((1, h2), lambda i: (0, 0)),
            pl.BlockSpec((h2, 1), lambda i: (0, 0)),
            pl.BlockSpec((1, 1), lambda i: (0, 0)),
        ],
        out_specs=pl.BlockSpec((G, 1), lambda i: (0, 0)),
        out_shape=jax.ShapeDtypeStruct((G, 1), jnp.float32),
    )(pool_parts, cnt_parts, w1, b1, w2, b2)


def kernel(x, edge_index, batch, W_in, b_in, conv_W, conv_b, ln_g, ln_b,
           W1, b1, W2, b2):
    src = edge_index[0]
    dst = edge_index[1]
    _sc_degree, _sc_edge_agg, _sc_pool = _sc_kernels()

    deg_parts = _sc_degree(dst)
    dinv = _tc_dinv(deg_parts)

    ones_scale = jnp.ones((N, 1), jnp.float32)
    h = _tc_mm(x, W_in, b_in.reshape(1, H), ones_scale)

    zero_bias = jnp.zeros((1, H), jnp.float32)
    for i in range(L):
        m2 = _tc_mm(h, conv_W[i], zero_bias, dinv)
        agg_parts = _sc_edge_agg(m2, edge_index)
        h = _tc_post(agg_parts, m2, dinv, conv_b[i].reshape(1, H),
                     ln_g[i].reshape(1, H), ln_b[i].reshape(1, H), h)

    pool_parts, cnt_parts = _sc_pool(h, batch)
    out = _tc_mlp(pool_parts, cnt_parts, W1, b1.reshape(1, -1), W2,
                  b2.reshape(1, 1))
    return out
